# Initial kernel scaffold; baseline (speedup 1.0000x reference)
import jax
import jax.numpy as jnp
from jax.experimental import pallas as pl


def kernel(x, edge_index, W1, b1, W2, b2):
    raise NotImplementedError("write your pallas kernel here")


# speedup vs baseline: 3.0350x; 3.0350x over previous
import functools

import jax
import jax.numpy as jnp
from jax import lax
from jax.experimental import pallas as pl
from jax.experimental.pallas import tpu as pltpu
from jax.experimental.pallas import tpu_sc as plsc

N_NODES = 10000
D_IN = 256
D_HID = 512

NC, NS = 2, 16
NW = NC * NS
KE = 128
CCH = 128
NCH = D_HID // CCH
N_PAD = 10240
RPT = N_PAD // NS
DEG_W = 128

_f32 = jnp.float32


def _sc_mesh():
    return plsc.VectorSubcoreMesh(core_axis_name="c", subcore_axis_name="s")


def _deg_body(e_pad, dst_hbm, ones_hbm, zeros_hbm, rows_hbm, out_hbm,
              ones_v, zbuf_v, idx_v, ridx_v, acc_sh):
    core = lax.axis_index("c")
    tile = lax.axis_index("s")
    pltpu.sync_copy(ones_hbm, ones_v)
    pltpu.sync_copy(zeros_hbm, zbuf_v)

    def zloop(j, _):
        pltpu.sync_copy(rows_hbm.at[pl.ds(tile * RPT + j * KE, KE)], ridx_v)
        pltpu.sync_copy(zbuf_v, acc_sh.at[ridx_v])
        return 0

    lax.fori_loop(0, RPT // KE, zloop, 0)
    plsc.subcore_barrier()

    ept = e_pad // NW
    base = (core * NS + tile) * ept

    def eloop(i, _):
        pltpu.sync_copy(dst_hbm.at[pl.ds(base + i * KE, KE)], idx_v)
        pltpu.sync_copy(ones_v, acc_sh.at[idx_v], add=True)
        return 0

    lax.fori_loop(0, ept // KE, eloop, 0)
    plsc.subcore_barrier()

    def rloop(j, _):
        r0 = tile * RPT + j * KE
        pltpu.sync_copy(rows_hbm.at[pl.ds(r0, KE)], ridx_v)
        pltpu.sync_copy(acc_sh.at[ridx_v], zbuf_v)
        pltpu.sync_copy(zbuf_v, out_hbm.at[core, pl.ds(r0, KE)])
        return 0

    lax.fori_loop(0, RPT // KE, rloop, 0)


def _deg_call(dst_p, ones, zeros, rows):
    e_pad = dst_p.shape[0]
    k = pl.kernel(
        functools.partial(_deg_body, e_pad),
        out_type=jax.ShapeDtypeStruct((NC, N_PAD, DEG_W), _f32),
        mesh=_sc_mesh(),
        scratch_types=[
            pltpu.VMEM((KE, DEG_W), _f32),
            pltpu.VMEM((KE, DEG_W), _f32),
            pltpu.VMEM((KE,), jnp.int32),
            pltpu.VMEM((KE,), jnp.int32),
            pltpu.VMEM_SHARED((N_PAD, DEG_W), _f32),
        ],
    )
    return k(dst_p, ones, zeros, rows)


def _agg_body(e_pad, p0, p1, p2, p3, src_hbm, dst_hbm, zeros_hbm, rows_hbm,
              out_hbm, rows_v, src_v, dst_v, ridx_v, sem, acc_sh):
    core = lax.axis_index("c")
    tile = lax.axis_index("s")

    eph = e_pad // NC
    ept = eph // NS
    base = core * eph + tile * ept
    nblk = ept // KE
    p_chunks = (p0, p1, p2, p3)

    for ch in range(NCH):
        p_hbm = p_chunks[ch]

        pltpu.sync_copy(zeros_hbm, rows_v)

        def zloop(j, _):
            pltpu.sync_copy(rows_hbm.at[pl.ds(tile * RPT + j * KE, KE)], ridx_v)
            pltpu.sync_copy(rows_v, acc_sh.at[ridx_v])
            return 0

        lax.fori_loop(0, RPT // KE, zloop, 0)
        plsc.subcore_barrier()

        def eloop(i, _):
            e0 = base + i * KE
            pltpu.sync_copy(src_hbm.at[pl.ds(e0, KE)], src_v)
            pltpu.sync_copy(dst_hbm.at[pl.ds(e0, KE)], dst_v)
            pltpu.async_copy(p_hbm.at[src_v], rows_v, sem).wait()
            pltpu.sync_copy(rows_v, acc_sh.at[dst_v], add=True)
            return 0

        lax.fori_loop(0, nblk, eloop, 0)
        plsc.subcore_barrier()

        def wloop(j, _):
            r0 = tile * RPT + j * KE
            pltpu.sync_copy(rows_hbm.at[pl.ds(r0, KE)], ridx_v)
            pltpu.sync_copy(acc_sh.at[ridx_v], rows_v)
            pltpu.sync_copy(rows_v, out_hbm.at[core, ch, pl.ds(r0, KE)])
            return 0

        lax.fori_loop(0, RPT // KE, wloop, 0)


@functools.lru_cache(maxsize=None)
def _agg_kernel(e_pad):
    return pl.kernel(
        functools.partial(_agg_body, e_pad),
        out_type=jax.ShapeDtypeStruct((NC, NCH, N_PAD, CCH), _f32),
        mesh=_sc_mesh(),
        scratch_types=[
            pltpu.VMEM((KE, CCH), _f32),
            pltpu.VMEM((KE,), jnp.int32),
            pltpu.VMEM((KE,), jnp.int32),
            pltpu.VMEM((KE,), jnp.int32),
            pltpu.SemaphoreType.DMA,
            pltpu.VMEM_SHARED((N_PAD, CCH), _f32),
        ],
    )


def _agg_call(p_t, src_p, dst_p, zeros, rows):
    k = _agg_kernel(src_p.shape[0])
    return k(p_t[0], p_t[1], p_t[2], p_t[3], src_p, dst_p, zeros, rows)


BN = 512


def _mm1_body(dis_ref, x_ref, w_ref, o_ref):
    h = jnp.dot(x_ref[...], w_ref[...], preferred_element_type=_f32)
    o_ref[0] = h * dis_ref[...]


def _mm1_call(dis_p, x_p, w1):
    nb = N_PAD // BN
    return pl.pallas_call(
        _mm1_body,
        grid=(NCH, nb),
        in_specs=[
            pl.BlockSpec((BN, 1), lambda c, i: (i, 0)),
            pl.BlockSpec((BN, D_IN), lambda c, i: (i, 0)),
            pl.BlockSpec((D_IN, CCH), lambda c, i: (0, c)),
        ],
        out_specs=pl.BlockSpec((1, BN, CCH), lambda c, i: (c, i, 0)),
        out_shape=jax.ShapeDtypeStruct((NCH, N_PAD, CCH), _f32),
    )(dis_p, x_p, w1)


def _mid_body(dis_ref, p1_ref, a_ref, b1_ref, w_ref, o_ref):
    a = a_ref[...]
    s = p1_ref[...] + a[0] + a[1]
    z = jnp.concatenate([s[c] for c in range(NCH)], axis=-1)
    z = jnp.maximum(z * dis_ref[...] + b1_ref[...], 0.0)
    o_ref[0] = jnp.dot(z, w_ref[...], preferred_element_type=_f32) * dis_ref[...]


def _mid_call(dis_p, p1_t, agg1, b1, w2):
    nb = N_PAD // BN
    return pl.pallas_call(
        _mid_body,
        grid=(NCH, nb),
        in_specs=[
            pl.BlockSpec((BN, 1), lambda c, i: (i, 0)),
            pl.BlockSpec((NCH, BN, CCH), lambda c, i: (0, i, 0)),
            pl.BlockSpec((NC, NCH, BN, CCH), lambda c, i: (0, 0, i, 0)),
            pl.BlockSpec((1, D_HID), lambda c, i: (0, 0)),
            pl.BlockSpec((D_HID, CCH), lambda c, i: (0, c)),
        ],
        out_specs=pl.BlockSpec((1, BN, CCH), lambda c, i: (c, i, 0)),
        out_shape=jax.ShapeDtypeStruct((NCH, N_PAD, CCH), _f32),
    )(dis_p, p1_t, agg1, b1, w2)


def _fin_body(dis_ref, p2_ref, a_ref, b2_ref, o_ref):
    a = a_ref[...]
    s = p2_ref[...] + a[0] + a[1]
    z = jnp.concatenate([s[c] for c in range(NCH)], axis=-1)
    o_ref[...] = z * dis_ref[...] + b2_ref[...]


def _fin_call(dis_p, p2_t, agg2, b2):
    nb = N_PAD // BN
    return pl.pallas_call(
        _fin_body,
        grid=(nb,),
        in_specs=[
            pl.BlockSpec((BN, 1), lambda i: (i, 0)),
            pl.BlockSpec((NCH, BN, CCH), lambda i: (0, i, 0)),
            pl.BlockSpec((NC, NCH, BN, CCH), lambda i: (0, 0, i, 0)),
            pl.BlockSpec((1, D_HID), lambda i: (0, 0)),
        ],
        out_specs=pl.BlockSpec((BN, D_HID), lambda i: (i, 0)),
        out_shape=jax.ShapeDtypeStruct((N_PAD, D_HID), _f32),
    )(dis_p, p2_t, agg2, b2)


def kernel(x, edge_index, W1, b1, W2, b2):
    n = x.shape[0]
    e = edge_index.shape[1]
    e_pad = ((e + NW * KE - 1) // (NW * KE)) * (NW * KE)

    x_p = jnp.zeros((N_PAD, D_IN), _f32).at[:n].set(x)
    src_p = jnp.zeros((e_pad,), jnp.int32).at[:e].set(edge_index[0])
    dst_p = jnp.full((e_pad,), n, jnp.int32).at[:e].set(edge_index[1])

    ones_deg = jnp.ones((KE, DEG_W), _f32)
    zeros_deg = jnp.zeros((KE, DEG_W), _f32)
    zeros_agg = jnp.zeros((KE, CCH), _f32)
    rows = jnp.arange(N_PAD, dtype=jnp.int32)

    degp = _deg_call(dst_p, ones_deg, zeros_deg, rows)
    deg = degp[0, :, 0] + degp[1, :, 0] + 1.0
    dis = lax.rsqrt(deg)
    dis_p = dis.reshape(N_PAD, 1)

    p1_t = _mm1_call(dis_p, x_p, W1)
    agg1 = _agg_call(p1_t, src_p, dst_p, zeros_agg, rows)

    p2_t = _mid_call(dis_p, p1_t, agg1, b1.reshape(1, D_HID), W2)
    agg2 = _agg_call(p2_t, src_p, dst_p, zeros_agg, rows)

    out_p = _fin_call(dis_p, p2_t, agg2, b2.reshape(1, D_HID))
    return out_p[:n]


# speedup vs baseline: 3.5962x; 1.1849x over previous
import functools

import jax
import jax.numpy as jnp
from jax import lax
from jax.experimental import pallas as pl
from jax.experimental.pallas import tpu as pltpu
from jax.experimental.pallas import tpu_sc as plsc

N_NODES = 10000
D_IN = 256
D_HID = 512

NC, NS = 2, 16
NW = NC * NS
KE = 128
CCH = 128
NCH = D_HID // CCH
N_PAD = 10240
RPT = N_PAD // NS
DEG_W = 128

_f32 = jnp.float32


def _sc_mesh():
    return plsc.VectorSubcoreMesh(core_axis_name="c", subcore_axis_name="s")


def _deg_body(e_pad, dst_hbm, ones_hbm, zeros_hbm, rows_hbm, out_hbm,
              ones_v, zbuf_v, idx_v, ridx_v, acc_sh):
    core = lax.axis_index("c")
    tile = lax.axis_index("s")
    pltpu.sync_copy(ones_hbm, ones_v)
    pltpu.sync_copy(zeros_hbm, zbuf_v)

    def zloop(j, _):
        pltpu.sync_copy(rows_hbm.at[pl.ds(tile * RPT + j * KE, KE)], ridx_v)
        pltpu.sync_copy(zbuf_v, acc_sh.at[ridx_v])
        return 0

    lax.fori_loop(0, RPT // KE, zloop, 0)
    plsc.subcore_barrier()

    ept = e_pad // NW
    base = (core * NS + tile) * ept

    def eloop(i, _):
        pltpu.sync_copy(dst_hbm.at[pl.ds(base + i * KE, KE)], idx_v)
        pltpu.sync_copy(ones_v, acc_sh.at[idx_v], add=True)
        return 0

    lax.fori_loop(0, ept // KE, eloop, 0)
    plsc.subcore_barrier()

    def rloop(j, _):
        r0 = tile * RPT + j * KE
        pltpu.sync_copy(rows_hbm.at[pl.ds(r0, KE)], ridx_v)
        pltpu.sync_copy(acc_sh.at[ridx_v], zbuf_v)
        pltpu.sync_copy(zbuf_v, out_hbm.at[core, pl.ds(r0, KE)])
        return 0

    lax.fori_loop(0, RPT // KE, rloop, 0)


def _deg_call(dst_p, ones, zeros, rows):
    e_pad = dst_p.shape[0]
    k = pl.kernel(
        functools.partial(_deg_body, e_pad),
        out_type=jax.ShapeDtypeStruct((NC, N_PAD, DEG_W), _f32),
        mesh=_sc_mesh(),
        scratch_types=[
            pltpu.VMEM((KE, DEG_W), _f32),
            pltpu.VMEM((KE, DEG_W), _f32),
            pltpu.VMEM((KE,), jnp.int32),
            pltpu.VMEM((KE,), jnp.int32),
            pltpu.VMEM_SHARED((N_PAD, DEG_W), _f32),
        ],
    )
    return k(dst_p, ones, zeros, rows)


def _agg_body(e_pad, p0, p1, p2, p3, src2_hbm, dst2_hbm, zeros_hbm, rows_hbm,
              out_hbm, rows_a, rows_b, sidx_v, didx_v, ridx_v, sem_a, sem_b,
              acc_sh):
    core = lax.axis_index("c")
    tile = lax.axis_index("s")

    eph = e_pad // NC
    ept = eph // NS
    nblk = ept // KE
    wid = core * NS + tile
    p_chunks = (p0, p1, p2, p3)

    pltpu.sync_copy(src2_hbm.at[wid], sidx_v)
    pltpu.sync_copy(dst2_hbm.at[wid], didx_v)

    for ch in range(NCH):
        p_hbm = p_chunks[ch]

        pltpu.sync_copy(zeros_hbm, rows_a)

        def zloop(j, _):
            pltpu.sync_copy(rows_hbm.at[pl.ds(tile * RPT + j * KE, KE)], ridx_v)
            pltpu.sync_copy(rows_a, acc_sh.at[ridx_v])
            return 0

        lax.fori_loop(0, RPT // KE, zloop, 0)
        plsc.subcore_barrier()

        pltpu.async_copy(p_hbm.at[sidx_v.at[0]], rows_a, sem_a)

        def eloop(j, _):
            i0 = 2 * j
            pltpu.make_async_copy(p_hbm.at[sidx_v.at[i0]], rows_a, sem_a).wait()
            pltpu.async_copy(p_hbm.at[sidx_v.at[i0 + 1]], rows_b, sem_b)
            pltpu.sync_copy(rows_a, acc_sh.at[didx_v.at[i0]], add=True)
            pltpu.make_async_copy(p_hbm.at[sidx_v.at[i0 + 1]], rows_b, sem_b).wait()

            @pl.when(i0 + 2 < nblk)
            def _():
                pltpu.async_copy(p_hbm.at[sidx_v.at[i0 + 2]], rows_a, sem_a)

            pltpu.sync_copy(rows_b, acc_sh.at[didx_v.at[i0 + 1]], add=True)
            return 0

        lax.fori_loop(0, nblk // 2, eloop, 0)
        plsc.subcore_barrier()

        def wloop(j, _):
            r0 = tile * RPT + j * KE
            pltpu.sync_copy(rows_hbm.at[pl.ds(r0, KE)], ridx_v)
            pltpu.sync_copy(acc_sh.at[ridx_v], rows_a)
            pltpu.sync_copy(rows_a, out_hbm.at[core, ch, pl.ds(r0, KE)])
            return 0

        lax.fori_loop(0, RPT // KE, wloop, 0)


@functools.lru_cache(maxsize=None)
def _agg_kernel(e_pad):
    nblk = e_pad // (NW * KE)
    return pl.kernel(
        functools.partial(_agg_body, e_pad),
        out_type=jax.ShapeDtypeStruct((NC, NCH, N_PAD, CCH), _f32),
        mesh=_sc_mesh(),
        scratch_types=[
            pltpu.VMEM((KE, CCH), _f32),
            pltpu.VMEM((KE, CCH), _f32),
            pltpu.VMEM((nblk, KE), jnp.int32),
            pltpu.VMEM((nblk, KE), jnp.int32),
            pltpu.VMEM((KE,), jnp.int32),
            pltpu.SemaphoreType.DMA,
            pltpu.SemaphoreType.DMA,
            pltpu.VMEM_SHARED((N_PAD, CCH), _f32),
        ],
    )


def _agg_call(p_t, src_p, dst_p, zeros, rows):
    e_pad = src_p.shape[0]
    k = _agg_kernel(e_pad)
    src2 = src_p.reshape(NW, e_pad // (NW * KE), KE)
    dst2 = dst_p.reshape(NW, e_pad // (NW * KE), KE)
    return k(p_t[0], p_t[1], p_t[2], p_t[3], src2, dst2, zeros, rows)


BN = 512


def _mm1_body(dis_ref, x_ref, w_ref, o_ref):
    h = jnp.dot(x_ref[...], w_ref[...], preferred_element_type=_f32)
    o_ref[0] = h * dis_ref[...]


def _mm1_call(dis_p, x_p, w1):
    nb = N_PAD // BN
    return pl.pallas_call(
        _mm1_body,
        grid=(NCH, nb),
        in_specs=[
            pl.BlockSpec((BN, 1), lambda c, i: (i, 0)),
            pl.BlockSpec((BN, D_IN), lambda c, i: (i, 0)),
            pl.BlockSpec((D_IN, CCH), lambda c, i: (0, c)),
        ],
        out_specs=pl.BlockSpec((1, BN, CCH), lambda c, i: (c, i, 0)),
        out_shape=jax.ShapeDtypeStruct((NCH, N_PAD, CCH), _f32),
    )(dis_p, x_p, w1)


def _mid_body(dis_ref, p1_ref, a_ref, b1_ref, w_ref, o_ref):
    a = a_ref[...]
    s = p1_ref[...] + a[0] + a[1]
    z = jnp.concatenate([s[c] for c in range(NCH)], axis=-1)
    z = jnp.maximum(z * dis_ref[...] + b1_ref[...], 0.0)
    o_ref[0] = jnp.dot(z, w_ref[...], preferred_element_type=_f32) * dis_ref[...]


def _mid_call(dis_p, p1_t, agg1, b1, w2):
    nb = N_PAD // BN
    return pl.pallas_call(
        _mid_body,
        grid=(NCH, nb),
        in_specs=[
            pl.BlockSpec((BN, 1), lambda c, i: (i, 0)),
            pl.BlockSpec((NCH, BN, CCH), lambda c, i: (0, i, 0)),
            pl.BlockSpec((NC, NCH, BN, CCH), lambda c, i: (0, 0, i, 0)),
            pl.BlockSpec((1, D_HID), lambda c, i: (0, 0)),
            pl.BlockSpec((D_HID, CCH), lambda c, i: (0, c)),
        ],
        out_specs=pl.BlockSpec((1, BN, CCH), lambda c, i: (c, i, 0)),
        out_shape=jax.ShapeDtypeStruct((NCH, N_PAD, CCH), _f32),
    )(dis_p, p1_t, agg1, b1, w2)


def _fin_body(dis_ref, p2_ref, a_ref, b2_ref, o_ref):
    a = a_ref[...]
    s = p2_ref[...] + a[0] + a[1]
    z = jnp.concatenate([s[c] for c in range(NCH)], axis=-1)
    o_ref[...] = z * dis_ref[...] + b2_ref[...]


def _fin_call(dis_p, p2_t, agg2, b2):
    nb = N_PAD // BN
    return pl.pallas_call(
        _fin_body,
        grid=(nb,),
        in_specs=[
            pl.BlockSpec((BN, 1), lambda i: (i, 0)),
            pl.BlockSpec((NCH, BN, CCH), lambda i: (0, i, 0)),
            pl.BlockSpec((NC, NCH, BN, CCH), lambda i: (0, 0, i, 0)),
            pl.BlockSpec((1, D_HID), lambda i: (0, 0)),
        ],
        out_specs=pl.BlockSpec((BN, D_HID), lambda i: (i, 0)),
        out_shape=jax.ShapeDtypeStruct((N_PAD, D_HID), _f32),
    )(dis_p, p2_t, agg2, b2)


def kernel(x, edge_index, W1, b1, W2, b2):
    n = x.shape[0]
    e = edge_index.shape[1]
    unit = 2 * NW * KE
    e_pad = ((e + unit - 1) // unit) * unit

    x_p = jnp.zeros((N_PAD, D_IN), _f32).at[:n].set(x)
    src_p = jnp.zeros((e_pad,), jnp.int32).at[:e].set(edge_index[0])
    dst_p = jnp.full((e_pad,), n, jnp.int32).at[:e].set(edge_index[1])

    ones_deg = jnp.ones((KE, DEG_W), _f32)
    zeros_deg = jnp.zeros((KE, DEG_W), _f32)
    zeros_agg = jnp.zeros((KE, CCH), _f32)
    rows = jnp.arange(N_PAD, dtype=jnp.int32)

    degp = _deg_call(dst_p, ones_deg, zeros_deg, rows)
    deg = degp[0, :, 0] + degp[1, :, 0] + 1.0
    dis = lax.rsqrt(deg)
    dis_p = dis.reshape(N_PAD, 1)

    p1_t = _mm1_call(dis_p, x_p, W1)
    agg1 = _agg_call(p1_t, src_p, dst_p, zeros_agg, rows)

    p2_t = _mid_call(dis_p, p1_t, agg1, b1.reshape(1, D_HID), W2)
    agg2 = _agg_call(p2_t, src_p, dst_p, zeros_agg, rows)

    out_p = _fin_call(dis_p, p2_t, agg2, b2.reshape(1, D_HID))
    return out_p[:n]


# speedup vs baseline: 3.6215x; 1.0070x over previous
import functools

import jax
import jax.numpy as jnp
from jax import lax
from jax.experimental import pallas as pl
from jax.experimental.pallas import tpu as pltpu
from jax.experimental.pallas import tpu_sc as plsc

N_NODES = 10000
D_IN = 256
D_HID = 512

NC, NS = 2, 16
NW = NC * NS
KE = 128
CCH = 128
NCH = D_HID // CCH
N_PAD = 10240
RPT = N_PAD // NS
DEG_W = 128

_f32 = jnp.float32


def _sc_mesh():
    return plsc.VectorSubcoreMesh(core_axis_name="c", subcore_axis_name="s")


def _deg_body(e_pad, dst_hbm, ones_hbm, zeros_hbm, rows_hbm, out_hbm,
              ones_v, zbuf_v, idx_v, ridx_v, acc_sh):
    core = lax.axis_index("c")
    tile = lax.axis_index("s")
    pltpu.sync_copy(ones_hbm, ones_v)
    pltpu.sync_copy(zeros_hbm, zbuf_v)

    def zloop(j, _):
        pltpu.sync_copy(rows_hbm.at[pl.ds(tile * RPT + j * KE, KE)], ridx_v)
        pltpu.sync_copy(zbuf_v, acc_sh.at[ridx_v])
        return 0

    lax.fori_loop(0, RPT // KE, zloop, 0)
    plsc.subcore_barrier()

    ept = e_pad // NW
    base = (core * NS + tile) * ept

    def eloop(i, _):
        pltpu.sync_copy(dst_hbm.at[pl.ds(base + i * KE, KE)], idx_v)
        pltpu.sync_copy(ones_v, acc_sh.at[idx_v], add=True)
        return 0

    lax.fori_loop(0, ept // KE, eloop, 0)
    plsc.subcore_barrier()

    def rloop(j, _):
        r0 = tile * RPT + j * KE
        pltpu.sync_copy(rows_hbm.at[pl.ds(r0, KE)], ridx_v)
        pltpu.sync_copy(acc_sh.at[ridx_v], zbuf_v)
        pltpu.sync_copy(zbuf_v, out_hbm.at[core, pl.ds(r0, KE)])
        return 0

    lax.fori_loop(0, RPT // KE, rloop, 0)


def _deg_call(dst_p, ones, zeros, rows):
    e_pad = dst_p.shape[0]
    k = pl.kernel(
        functools.partial(_deg_body, e_pad),
        out_type=jax.ShapeDtypeStruct((NC, N_PAD, DEG_W), _f32),
        mesh=_sc_mesh(),
        scratch_types=[
            pltpu.VMEM((KE, DEG_W), _f32),
            pltpu.VMEM((KE, DEG_W), _f32),
            pltpu.VMEM((KE,), jnp.int32),
            pltpu.VMEM((KE,), jnp.int32),
            pltpu.VMEM_SHARED((N_PAD, DEG_W), _f32),
        ],
    )
    return k(dst_p, ones, zeros, rows)


def _agg_body(e_pad, p0, p1, p2, p3, src2_hbm, dst2_hbm, zeros_hbm, rows_hbm,
              out_hbm, rows_a, rows_b, sidx_v, didx_v, ridx_v, sem_a, sem_b,
              sem_sa, sem_sb, acc_sh):
    core = lax.axis_index("c")
    tile = lax.axis_index("s")

    eph = e_pad // NC
    ept = eph // NS
    nblk = ept // KE
    wid = core * NS + tile
    p_chunks = (p0, p1, p2, p3)

    pltpu.sync_copy(src2_hbm.at[wid], sidx_v)
    pltpu.sync_copy(dst2_hbm.at[wid], didx_v)

    for ch in range(NCH):
        p_hbm = p_chunks[ch]

        pltpu.sync_copy(zeros_hbm, rows_a)

        def zloop(j, _):
            pltpu.sync_copy(rows_hbm.at[pl.ds(tile * RPT + j * KE, KE)], ridx_v)
            pltpu.sync_copy(rows_a, acc_sh.at[ridx_v])
            return 0

        lax.fori_loop(0, RPT // KE, zloop, 0)
        plsc.subcore_barrier()

        pltpu.async_copy(p_hbm.at[sidx_v.at[0]], rows_a, sem_a)
        pltpu.async_copy(p_hbm.at[sidx_v.at[1]], rows_b, sem_b)

        def eloop(j, _):
            i0 = 2 * j
            pltpu.make_async_copy(p_hbm.at[sidx_v.at[i0]], rows_a, sem_a).wait()
            pltpu.async_copy(rows_a, acc_sh.at[didx_v.at[i0]], sem_sa, add=True)
            pltpu.make_async_copy(p_hbm.at[sidx_v.at[i0 + 1]], rows_b, sem_b).wait()
            pltpu.async_copy(rows_b, acc_sh.at[didx_v.at[i0 + 1]], sem_sb, add=True)

            @pl.when(i0 + 2 < nblk)
            def _():
                pltpu.make_async_copy(
                    rows_a, acc_sh.at[didx_v.at[i0]], sem_sa).wait()
                pltpu.async_copy(p_hbm.at[sidx_v.at[i0 + 2]], rows_a, sem_a)
                pltpu.make_async_copy(
                    rows_b, acc_sh.at[didx_v.at[i0 + 1]], sem_sb).wait()
                pltpu.async_copy(p_hbm.at[sidx_v.at[i0 + 3]], rows_b, sem_b)

            return 0

        lax.fori_loop(0, nblk // 2, eloop, 0)
        pltpu.make_async_copy(rows_a, acc_sh.at[didx_v.at[nblk - 2]], sem_sa).wait()
        pltpu.make_async_copy(rows_b, acc_sh.at[didx_v.at[nblk - 1]], sem_sb).wait()
        plsc.subcore_barrier()

        def wloop(j, _):
            r0 = tile * RPT + j * KE
            pltpu.sync_copy(rows_hbm.at[pl.ds(r0, KE)], ridx_v)
            pltpu.sync_copy(acc_sh.at[ridx_v], rows_a)
            pltpu.sync_copy(rows_a, out_hbm.at[core, ch, pl.ds(r0, KE)])
            return 0

        lax.fori_loop(0, RPT // KE, wloop, 0)


@functools.lru_cache(maxsize=None)
def _agg_kernel(e_pad):
    nblk = e_pad // (NW * KE)
    return pl.kernel(
        functools.partial(_agg_body, e_pad),
        out_type=jax.ShapeDtypeStruct((NC, NCH, N_PAD, CCH), _f32),
        mesh=_sc_mesh(),
        scratch_types=[
            pltpu.VMEM((KE, CCH), _f32),
            pltpu.VMEM((KE, CCH), _f32),
            pltpu.VMEM((nblk, KE), jnp.int32),
            pltpu.VMEM((nblk, KE), jnp.int32),
            pltpu.VMEM((KE,), jnp.int32),
            pltpu.SemaphoreType.DMA,
            pltpu.SemaphoreType.DMA,
            pltpu.SemaphoreType.DMA,
            pltpu.SemaphoreType.DMA,
            pltpu.VMEM_SHARED((N_PAD, CCH), _f32),
        ],
    )


def _agg_call(p_t, src_p, dst_p, zeros, rows):
    e_pad = src_p.shape[0]
    k = _agg_kernel(e_pad)
    src2 = src_p.reshape(NW, e_pad // (NW * KE), KE)
    dst2 = dst_p.reshape(NW, e_pad // (NW * KE), KE)
    return k(p_t[0], p_t[1], p_t[2], p_t[3], src2, dst2, zeros, rows)


BN = 512


def _mm1_body(dis_ref, x_ref, w_ref, o_ref):
    h = jnp.dot(x_ref[...], w_ref[...], preferred_element_type=_f32)
    o_ref[0] = h * dis_ref[...]


def _mm1_call(dis_p, x_p, w1):
    nb = N_PAD // BN
    return pl.pallas_call(
        _mm1_body,
        grid=(NCH, nb),
        in_specs=[
            pl.BlockSpec((BN, 1), lambda c, i: (i, 0)),
            pl.BlockSpec((BN, D_IN), lambda c, i: (i, 0)),
            pl.BlockSpec((D_IN, CCH), lambda c, i: (0, c)),
        ],
        out_specs=pl.BlockSpec((1, BN, CCH), lambda c, i: (c, i, 0)),
        out_shape=jax.ShapeDtypeStruct((NCH, N_PAD, CCH), _f32),
    )(dis_p, x_p, w1)


def _mid_body(dis_ref, p1_ref, a_ref, b1_ref, w_ref, o_ref):
    a = a_ref[...]
    s = p1_ref[...] + a[0] + a[1]
    z = jnp.concatenate([s[c] for c in range(NCH)], axis=-1)
    z = jnp.maximum(z * dis_ref[...] + b1_ref[...], 0.0)
    o_ref[0] = jnp.dot(z, w_ref[...], preferred_element_type=_f32) * dis_ref[...]


def _mid_call(dis_p, p1_t, agg1, b1, w2):
    nb = N_PAD // BN
    return pl.pallas_call(
        _mid_body,
        grid=(NCH, nb),
        in_specs=[
            pl.BlockSpec((BN, 1), lambda c, i: (i, 0)),
            pl.BlockSpec((NCH, BN, CCH), lambda c, i: (0, i, 0)),
            pl.BlockSpec((NC, NCH, BN, CCH), lambda c, i: (0, 0, i, 0)),
            pl.BlockSpec((1, D_HID), lambda c, i: (0, 0)),
            pl.BlockSpec((D_HID, CCH), lambda c, i: (0, c)),
        ],
        out_specs=pl.BlockSpec((1, BN, CCH), lambda c, i: (c, i, 0)),
        out_shape=jax.ShapeDtypeStruct((NCH, N_PAD, CCH), _f32),
    )(dis_p, p1_t, agg1, b1, w2)


def _fin_body(dis_ref, p2_ref, a_ref, b2_ref, o_ref):
    a = a_ref[...]
    s = p2_ref[...] + a[0] + a[1]
    z = jnp.concatenate([s[c] for c in range(NCH)], axis=-1)
    o_ref[...] = z * dis_ref[...] + b2_ref[...]


def _fin_call(dis_p, p2_t, agg2, b2):
    nb = N_PAD // BN
    return pl.pallas_call(
        _fin_body,
        grid=(nb,),
        in_specs=[
            pl.BlockSpec((BN, 1), lambda i: (i, 0)),
            pl.BlockSpec((NCH, BN, CCH), lambda i: (0, i, 0)),
            pl.BlockSpec((NC, NCH, BN, CCH), lambda i: (0, 0, i, 0)),
            pl.BlockSpec((1, D_HID), lambda i: (0, 0)),
        ],
        out_specs=pl.BlockSpec((BN, D_HID), lambda i: (i, 0)),
        out_shape=jax.ShapeDtypeStruct((N_PAD, D_HID), _f32),
    )(dis_p, p2_t, agg2, b2)


def kernel(x, edge_index, W1, b1, W2, b2):
    n = x.shape[0]
    e = edge_index.shape[1]
    unit = 2 * NW * KE
    e_pad = ((e + unit - 1) // unit) * unit

    x_p = jnp.zeros((N_PAD, D_IN), _f32).at[:n].set(x)
    src_p = jnp.zeros((e_pad,), jnp.int32).at[:e].set(edge_index[0])
    dst_p = jnp.full((e_pad,), n, jnp.int32).at[:e].set(edge_index[1])

    ones_deg = jnp.ones((KE, DEG_W), _f32)
    zeros_deg = jnp.zeros((KE, DEG_W), _f32)
    zeros_agg = jnp.zeros((KE, CCH), _f32)
    rows = jnp.arange(N_PAD, dtype=jnp.int32)

    degp = _deg_call(dst_p, ones_deg, zeros_deg, rows)
    deg = degp[0, :, 0] + degp[1, :, 0] + 1.0
    dis = lax.rsqrt(deg)
    dis_p = dis.reshape(N_PAD, 1)

    p1_t = _mm1_call(dis_p, x_p, W1)
    agg1 = _agg_call(p1_t, src_p, dst_p, zeros_agg, rows)

    p2_t = _mid_call(dis_p, p1_t, agg1, b1.reshape(1, D_HID), W2)
    agg2 = _agg_call(p2_t, src_p, dst_p, zeros_agg, rows)

    out_p = _fin_call(dis_p, p2_t, agg2, b2.reshape(1, D_HID))
    return out_p[:n]
